# Initial kernel scaffold; baseline (speedup 1.0000x reference)
import jax
import jax.numpy as jnp
from jax.experimental import pallas as pl


def kernel(x, W_enc, W_dec, bias_pre):
    raise NotImplementedError("write your pallas kernel here")


# speedup vs baseline: 12.9356x; 12.9356x over previous
import jax
import jax.numpy as jnp
from jax import lax
from jax.experimental import pallas as pl
from jax.experimental.pallas import tpu as pltpu

_INTERPRET = False

K_TOP = 32
K_DEAD = 64

_SWEEPS = 11
_GT_ROWS = 128
_DT_ROWS = 64
_MM_ROWS = 512
_MM_COLS = 2048
_R_ROWS = 256


def _mono_u32(f):
    b = lax.bitcast_convert_type(f, jnp.uint32)
    neg = (b >> jnp.uint32(31)) == jnp.uint32(1)
    return jnp.where(neg, ~b, b | jnp.uint32(0x80000000))


def _mm_body(x_ref, w_ref, b_ref, lin_ref):
    xb = x_ref[...] - b_ref[...]
    lin_ref[...] = lax.dot_general(
        xb, w_ref[...], (((1,), (1,)), ((), ())),
        preferred_element_type=jnp.float32)


def _compute_lin(x, W_enc, bias2d):
    bs, d_in = x.shape
    sd = W_enc.shape[0]
    gr, gc = bs // _MM_ROWS, sd // _MM_COLS
    return pl.pallas_call(
        _mm_body,
        grid=(gr, gc),
        in_specs=[
            pl.BlockSpec((_MM_ROWS, d_in), lambda r, c: (r, 0)),
            pl.BlockSpec((_MM_COLS, d_in), lambda r, c: (c, 0)),
            pl.BlockSpec((1, d_in), lambda r, c: (0, 0)),
        ],
        out_specs=pl.BlockSpec((_MM_ROWS, _MM_COLS), lambda r, c: (r, c)),
        out_shape=jax.ShapeDtypeStruct((bs, sd), jnp.float32),
        interpret=_INTERPRET,
    )(x, W_enc, bias2d)


def _make_gth_body(rank, n_tiles):
    def body(lin_ref, t_ref, bounds, counts):
        s = pl.program_id(0)
        t = pl.program_id(1)

        @pl.when((s == 0) & (t == 0))
        def _():
            bounds[0] = jnp.uint32(0)
            bounds[1] = jnp.uint32(0xFFFFFFFF)

        @pl.when(t == 0)
        def _():
            for e in range(16):
                counts[e] = jnp.int32(0)

        lo = bounds[0]
        hi = bounds[1]
        step = jnp.maximum((hi - lo) // jnp.uint32(16), jnp.uint32(1))
        u = _mono_u32(lin_ref[...])
        for e in range(16):
            edge = lo + step * jnp.uint32(e)
            counts[e] = counts[e] + jnp.sum((u >= edge).astype(jnp.int32))

        @pl.when(t == n_tiles - 1)
        def _():
            j = jnp.uint32(0)
            for e in range(1, 16):
                j = jnp.where(counts[e] >= rank, jnp.uint32(e), j)
            newlo = lo + step * j
            newhi = jnp.where(j == jnp.uint32(15), hi,
                              jnp.minimum(hi, lo + step * (j + jnp.uint32(1))))
            bounds[0] = newlo
            bounds[1] = newhi

            @pl.when(s == _SWEEPS - 1)
            def _():
                t_ref[0, 0] = newlo

    return body


def _global_threshold(lin, rank):
    bs, sd = lin.shape
    n_tiles = bs // _GT_ROWS
    return pl.pallas_call(
        _make_gth_body(rank, n_tiles),
        grid=(_SWEEPS, n_tiles),
        in_specs=[pl.BlockSpec((_GT_ROWS, sd), lambda s, t: (t, 0))],
        out_specs=pl.BlockSpec(memory_space=pltpu.SMEM),
        out_shape=jax.ShapeDtypeStruct((1, 1), jnp.uint32),
        scratch_shapes=[
            pltpu.SMEM((2,), jnp.uint32),
            pltpu.SMEM((16,), jnp.int32),
        ],
        interpret=_INTERPRET,
    )(lin)


def _recon_body(t_ref, lin_ref, w_ref, b_ref, recon_ref, cm_ref):
    c = pl.program_id(1)
    nc = pl.num_programs(1)
    t_u = t_ref[0, 0]
    linb = lin_ref[...]
    sel = _mono_u32(linb) >= t_u
    y = jnp.where(sel, linb, 0.0)
    part = lax.dot_general(
        y, w_ref[...], (((1,), (1,)), ((), ())),
        preferred_element_type=jnp.float32)

    @pl.when(c == 0)
    def _():
        recon_ref[...] = jnp.zeros_like(recon_ref)

    recon_ref[...] += part

    @pl.when(c == nc - 1)
    def _():
        recon_ref[...] += b_ref[...]

    cm_ref[...] = jnp.max(sel.astype(jnp.float32), axis=0)[None, None, :]


def _recon_and_colmask(t_u, lin, W_dec, bias2d):
    bs, sd = lin.shape
    d_in = W_dec.shape[0]
    gr, gc = bs // _R_ROWS, sd // _MM_COLS
    return pl.pallas_call(
        _recon_body,
        grid=(gr, gc),
        in_specs=[
            pl.BlockSpec(memory_space=pltpu.SMEM),
            pl.BlockSpec((_R_ROWS, _MM_COLS), lambda r, c: (r, c)),
            pl.BlockSpec((d_in, _MM_COLS), lambda r, c: (0, c)),
            pl.BlockSpec((1, d_in), lambda r, c: (0, 0)),
        ],
        out_specs=[
            pl.BlockSpec((_R_ROWS, d_in), lambda r, c: (r, 0)),
            pl.BlockSpec((1, 1, _MM_COLS), lambda r, c: (r, 0, c)),
        ],
        out_shape=[
            jax.ShapeDtypeStruct((bs, d_in), jnp.float32),
            jax.ShapeDtypeStruct((gr, 1, sd), jnp.float32),
        ],
        interpret=_INTERPRET,
    )(t_u, lin, W_dec, bias2d)


def _deadth_body(cm_ref, lin_ref, trow_ref, wu_ref):
    cm = jnp.max(cm_ref[...], axis=(0, 1))
    dead = (cm == 0.0)[None, :]
    u = _mono_u32(lin_ref[...])
    wu_ref[...] = jnp.where(dead, u, jnp.uint32(0))
    n = lin_ref.shape[0]
    lo0 = jnp.zeros((n, 1), jnp.uint32)
    hi0 = jnp.full((n, 1), jnp.uint32(0xFFFFFFFF))

    def body(_, carry):
        lo, hi = carry
        mid = lo + (hi - lo) // jnp.uint32(2)
        cnt = jnp.sum((wu_ref[...] >= mid).astype(jnp.int32),
                      axis=1, keepdims=True)
        ge = cnt >= K_DEAD
        return (jnp.where(ge, mid, lo), jnp.where(ge, hi, mid))

    lo, hi = lax.fori_loop(0, 33, body, (lo0, hi0))
    trow_ref[...] = lo


def _dead_thresholds(cm_part, lin):
    bs, sd = lin.shape
    g = bs // _DT_ROWS
    nrb = cm_part.shape[0]
    return pl.pallas_call(
        _deadth_body,
        grid=(g,),
        in_specs=[
            pl.BlockSpec((nrb, 1, sd), lambda r: (0, 0, 0)),
            pl.BlockSpec((_DT_ROWS, sd), lambda r: (r, 0)),
        ],
        out_specs=pl.BlockSpec((_DT_ROWS, 1), lambda r: (r, 0)),
        out_shape=jax.ShapeDtypeStruct((bs, 1), jnp.uint32),
        scratch_shapes=[pltpu.VMEM((_DT_ROWS, sd), jnp.uint32)],
        interpret=_INTERPRET,
    )(cm_part, lin)


def _deadrecon_body(cm_ref, trow_ref, lin_ref, w_ref, out_ref):
    c = pl.program_id(1)
    cm = jnp.max(cm_ref[...], axis=(0, 1))
    dead = (cm == 0.0)[None, :]
    t = trow_ref[...]
    linb = lin_ref[...]
    u = _mono_u32(linb)
    y = jnp.where(dead & (u >= t), linb, 0.0)
    part = lax.dot_general(
        y, w_ref[...], (((1,), (1,)), ((), ())),
        preferred_element_type=jnp.float32)

    @pl.when(c == 0)
    def _():
        out_ref[...] = jnp.zeros_like(out_ref)

    out_ref[...] += part


def _dead_recon(cm_part, t_row, lin, W_dec):
    bs, sd = lin.shape
    d_in = W_dec.shape[0]
    gr, gc = bs // _R_ROWS, sd // _MM_COLS
    nrb = cm_part.shape[0]
    return pl.pallas_call(
        _deadrecon_body,
        grid=(gr, gc),
        in_specs=[
            pl.BlockSpec((nrb, 1, _MM_COLS), lambda r, c: (0, 0, c)),
            pl.BlockSpec((_R_ROWS, 1), lambda r, c: (r, 0)),
            pl.BlockSpec((_R_ROWS, _MM_COLS), lambda r, c: (r, c)),
            pl.BlockSpec((d_in, _MM_COLS), lambda r, c: (0, c)),
        ],
        out_specs=pl.BlockSpec((_R_ROWS, d_in), lambda r, c: (r, 0)),
        out_shape=jax.ShapeDtypeStruct((bs, d_in), jnp.float32),
        interpret=_INTERPRET,
    )(cm_part, t_row, lin, W_dec)


def kernel(x, W_enc, W_dec, bias_pre):
    bs = x.shape[0]
    bias2d = bias_pre.reshape(1, -1)
    lin = _compute_lin(x, W_enc, bias2d)
    t_u = _global_threshold(lin, K_TOP * bs)
    recon, cm_part = _recon_and_colmask(t_u, lin, W_dec, bias2d)
    t_row = _dead_thresholds(cm_part, lin)
    dead_recon = _dead_recon(cm_part, t_row, lin, W_dec)
    return (recon, dead_recon)


# speedup vs baseline: 14.8437x; 1.1475x over previous
import jax
import jax.numpy as jnp
from jax import lax
from jax.experimental import pallas as pl
from jax.experimental.pallas import tpu as pltpu

_INTERPRET = False

K_TOP = 32
K_DEAD = 64

_SWEEPS = 9
_GT_ROWS = 128
_DT_ROWS = 64
_MM_ROWS = 512
_MM_COLS = 2048
_R_ROWS = 256


def _mono_u32(f):
    b = lax.bitcast_convert_type(f, jnp.uint32)
    neg = (b >> jnp.uint32(31)) == jnp.uint32(1)
    return jnp.where(neg, ~b, b | jnp.uint32(0x80000000))


def _mm_body(x_ref, w_ref, b_ref, lin_ref):
    xb = x_ref[...] - b_ref[...]
    lin_ref[...] = lax.dot_general(
        xb, w_ref[...], (((1,), (1,)), ((), ())),
        preferred_element_type=jnp.float32)


def _compute_lin(x, W_enc, bias2d):
    bs, d_in = x.shape
    sd = W_enc.shape[0]
    gr, gc = bs // _MM_ROWS, sd // _MM_COLS
    return pl.pallas_call(
        _mm_body,
        grid=(gr, gc),
        in_specs=[
            pl.BlockSpec((_MM_ROWS, d_in), lambda r, c: (r, 0)),
            pl.BlockSpec((_MM_COLS, d_in), lambda r, c: (c, 0)),
            pl.BlockSpec((1, d_in), lambda r, c: (0, 0)),
        ],
        out_specs=pl.BlockSpec((_MM_ROWS, _MM_COLS), lambda r, c: (r, c)),
        out_shape=jax.ShapeDtypeStruct((bs, sd), jnp.float32),
        interpret=_INTERPRET,
    )(x, W_enc, bias2d)


def _make_gth_body(rank, n_tiles):
    def body(lin_ref, t_ref, bounds, counts):
        s = pl.program_id(0)
        t = pl.program_id(1)

        @pl.when((s == 0) & (t == 0))
        def _():
            bounds[0] = jnp.uint32(0)
            bounds[1] = jnp.uint32(0xFFFFFFFF)

        @pl.when(t == 0)
        def _():
            for e in range(16):
                counts[e] = jnp.int32(0)

        lo = bounds[0]
        hi = bounds[1]
        step = jnp.maximum((hi - lo) // jnp.uint32(16), jnp.uint32(1))
        u = _mono_u32(lin_ref[...])
        for e in range(16):
            edge = lo + step * jnp.uint32(e)
            counts[e] = counts[e] + jnp.sum((u >= edge).astype(jnp.int32))

        @pl.when(t == n_tiles - 1)
        def _():
            j = jnp.uint32(0)
            for e in range(1, 16):
                j = jnp.where(counts[e] >= rank, jnp.uint32(e), j)
            newlo = lo + step * j
            newhi = jnp.where(j == jnp.uint32(15), hi,
                              jnp.minimum(hi, lo + step * (j + jnp.uint32(1))))
            bounds[0] = newlo
            bounds[1] = newhi

            @pl.when(s == _SWEEPS - 1)
            def _():
                t_ref[0, 0] = newlo

    return body


def _global_threshold(lin, rank):
    bs, sd = lin.shape
    n_tiles = bs // _GT_ROWS
    return pl.pallas_call(
        _make_gth_body(rank, n_tiles),
        grid=(_SWEEPS, n_tiles),
        in_specs=[pl.BlockSpec((_GT_ROWS, sd), lambda s, t: (t, 0))],
        out_specs=pl.BlockSpec(memory_space=pltpu.SMEM),
        out_shape=jax.ShapeDtypeStruct((1, 1), jnp.uint32),
        scratch_shapes=[
            pltpu.SMEM((2,), jnp.uint32),
            pltpu.SMEM((16,), jnp.int32),
        ],
        interpret=_INTERPRET,
    )(lin)


def _recon_body(t_ref, lin_ref, w_ref, b_ref, recon_ref, cm_ref):
    c = pl.program_id(1)
    nc = pl.num_programs(1)
    t_u = t_ref[0, 0]
    linb = lin_ref[...]
    sel = _mono_u32(linb) >= t_u
    y = jnp.where(sel, linb, 0.0)
    part = lax.dot_general(
        y, w_ref[...], (((1,), (1,)), ((), ())),
        preferred_element_type=jnp.float32)

    @pl.when(c == 0)
    def _():
        recon_ref[...] = jnp.zeros_like(recon_ref)

    recon_ref[...] += part

    @pl.when(c == nc - 1)
    def _():
        recon_ref[...] += b_ref[...]

    cm_ref[...] = jnp.max(sel.astype(jnp.float32), axis=0)[None, None, :]


def _recon_and_colmask(t_u, lin, W_dec, bias2d):
    bs, sd = lin.shape
    d_in = W_dec.shape[0]
    gr, gc = bs // _R_ROWS, sd // _MM_COLS
    return pl.pallas_call(
        _recon_body,
        grid=(gr, gc),
        in_specs=[
            pl.BlockSpec(memory_space=pltpu.SMEM),
            pl.BlockSpec((_R_ROWS, _MM_COLS), lambda r, c: (r, c)),
            pl.BlockSpec((d_in, _MM_COLS), lambda r, c: (0, c)),
            pl.BlockSpec((1, d_in), lambda r, c: (0, 0)),
        ],
        out_specs=[
            pl.BlockSpec((_R_ROWS, d_in), lambda r, c: (r, 0)),
            pl.BlockSpec((1, 1, _MM_COLS), lambda r, c: (r, 0, c)),
        ],
        out_shape=[
            jax.ShapeDtypeStruct((bs, d_in), jnp.float32),
            jax.ShapeDtypeStruct((gr, 1, sd), jnp.float32),
        ],
        interpret=_INTERPRET,
    )(t_u, lin, W_dec, bias2d)


def _deadth_body(cm_ref, lin_ref, trow_ref, wu_ref):
    cm = jnp.max(cm_ref[...], axis=(0, 1))
    dead = (cm == 0.0)[None, :]
    u = _mono_u32(lin_ref[...])
    wu_ref[...] = jnp.where(dead, u, jnp.uint32(0))
    n = lin_ref.shape[0]
    lo0 = jnp.zeros((n, 1), jnp.uint32)
    hi0 = jnp.full((n, 1), jnp.uint32(0xFFFFFFFF))

    def body(_, carry):
        lo, hi = carry
        mid = lo + (hi - lo) // jnp.uint32(2)
        cnt = jnp.sum((wu_ref[...] >= mid).astype(jnp.int32),
                      axis=1, keepdims=True)
        ge = cnt >= K_DEAD
        return (jnp.where(ge, mid, lo), jnp.where(ge, hi, mid))

    lo, hi = lax.fori_loop(0, 33, body, (lo0, hi0))
    trow_ref[...] = lo


def _dead_thresholds(cm_part, lin):
    bs, sd = lin.shape
    g = bs // _DT_ROWS
    nrb = cm_part.shape[0]
    return pl.pallas_call(
        _deadth_body,
        grid=(g,),
        in_specs=[
            pl.BlockSpec((nrb, 1, sd), lambda r: (0, 0, 0)),
            pl.BlockSpec((_DT_ROWS, sd), lambda r: (r, 0)),
        ],
        out_specs=pl.BlockSpec((_DT_ROWS, 1), lambda r: (r, 0)),
        out_shape=jax.ShapeDtypeStruct((bs, 1), jnp.uint32),
        scratch_shapes=[pltpu.VMEM((_DT_ROWS, sd), jnp.uint32)],
        interpret=_INTERPRET,
    )(cm_part, lin)


def _deadrecon_body(cm_ref, trow_ref, lin_ref, w_ref, out_ref):
    c = pl.program_id(1)
    cm = jnp.max(cm_ref[...], axis=(0, 1))
    dead = (cm == 0.0)[None, :]
    t = trow_ref[...]
    linb = lin_ref[...]
    u = _mono_u32(linb)
    y = jnp.where(dead & (u >= t), linb, 0.0)
    part = lax.dot_general(
        y, w_ref[...], (((1,), (1,)), ((), ())),
        preferred_element_type=jnp.float32)

    @pl.when(c == 0)
    def _():
        out_ref[...] = jnp.zeros_like(out_ref)

    out_ref[...] += part


def _dead_recon(cm_part, t_row, lin, W_dec):
    bs, sd = lin.shape
    d_in = W_dec.shape[0]
    gr, gc = bs // _R_ROWS, sd // _MM_COLS
    nrb = cm_part.shape[0]
    return pl.pallas_call(
        _deadrecon_body,
        grid=(gr, gc),
        in_specs=[
            pl.BlockSpec((nrb, 1, _MM_COLS), lambda r, c: (0, 0, c)),
            pl.BlockSpec((_R_ROWS, 1), lambda r, c: (r, 0)),
            pl.BlockSpec((_R_ROWS, _MM_COLS), lambda r, c: (r, c)),
            pl.BlockSpec((d_in, _MM_COLS), lambda r, c: (0, c)),
        ],
        out_specs=pl.BlockSpec((_R_ROWS, d_in), lambda r, c: (r, 0)),
        out_shape=jax.ShapeDtypeStruct((bs, d_in), jnp.float32),
        interpret=_INTERPRET,
    )(cm_part, t_row, lin, W_dec)


def kernel(x, W_enc, W_dec, bias_pre):
    bs = x.shape[0]
    bias2d = bias_pre.reshape(1, -1)
    lin = _compute_lin(x, W_enc, bias2d)
    t_u = _global_threshold(lin, K_TOP * bs)
    recon, cm_part = _recon_and_colmask(t_u, lin, W_dec, bias2d)
    t_row = _dead_thresholds(cm_part, lin)
    dead_recon = _dead_recon(cm_part, t_row, lin, W_dec)
    return (recon, dead_recon)
